# Initial kernel scaffold; baseline (speedup 1.0000x reference)
import jax
import jax.numpy as jnp
from jax.experimental import pallas as pl


def kernel(x, lut):
    raise NotImplementedError("write your pallas kernel here")


# speedup vs baseline: 1.3145x; 1.3145x over previous
import functools

import jax
import jax.numpy as jnp
from jax import lax
from jax.experimental import pallas as pl
from jax.experimental.pallas import tpu as pltpu
from jax.experimental.pallas import tpu_sc as plsc

BATCH = 16384
HIST = 50
D = 32
R = BATCH * HIST

NC = 2
NS = 16
NW = NC * NS
RPW = R // NW
CHUNK = 128
NCH = RPW // CHUNK
NBUF = 8
NR = NCH // NBUF

_mesh = plsc.VectorSubcoreMesh(core_axis_name="c", subcore_axis_name="s")


@functools.partial(
    pl.kernel,
    mesh=_mesh,
    out_type=jax.ShapeDtypeStruct((NW * NCH, CHUNK, D), jnp.float32),
    scratch_types=[
        pltpu.VMEM((NCH, CHUNK), jnp.int32),
        pltpu.VMEM((NBUF, CHUNK, D), jnp.float32),
        pltpu.SemaphoreType.DMA,
        pltpu.SemaphoreType.DMA,
    ],
    compiler_params=pltpu.CompilerParams(use_tc_tiling_on_sc=False),
)
def _emb_lookup(idx_hbm, table_hbm, out_hbm, idx_v, rows_v, sem_g, sem_o):
    wid = lax.axis_index("s") * NC + lax.axis_index("c")
    base = wid * NCH
    pltpu.sync_copy(idx_hbm.at[wid], idx_v)

    def round_body(g, carry):
        gathers = []
        for b in range(NBUF):
            j = g * NBUF + b
            c = pltpu.make_async_copy(
                table_hbm.at[idx_v.at[j]], rows_v.at[b], sem_g)
            c.start()
            gathers.append(c)
        outs = []
        for b in range(NBUF):
            j = g * NBUF + b
            gathers[b].wait()
            oc = pltpu.make_async_copy(
                rows_v.at[b], out_hbm.at[base + j], sem_o)
            oc.start()
            outs.append(oc)
        for oc in outs:
            oc.wait()
        return carry

    lax.fori_loop(0, NR, round_body, 0)


def kernel(x, lut):
    idx = x.reshape(NW, NCH, CHUNK)
    out = _emb_lookup(idx, lut)
    return out.reshape(BATCH, HIST, D)


# speedup vs baseline: 1.8194x; 1.3841x over previous
import functools

import jax
import jax.numpy as jnp
from jax import lax
from jax.experimental import pallas as pl
from jax.experimental.pallas import tpu as pltpu
from jax.experimental.pallas import tpu_sc as plsc

BATCH = 16384
HIST = 50
D = 32

NC = 2
NS = 16
NW = NC * NS
BPW = BATCH // NW
CHUNK = 128
NBLK = BPW // CHUNK
NUNIT = NBLK * HIST
NBUF = 8
NR = NUNIT // NBUF

_mesh = plsc.VectorSubcoreMesh(core_axis_name="c", subcore_axis_name="s")


@functools.partial(
    pl.kernel,
    mesh=_mesh,
    out_type=jax.ShapeDtypeStruct((BATCH, HIST, D), jnp.float32),
    scratch_types=[
        pltpu.VMEM((HIST, BPW), jnp.int32),
        pltpu.VMEM((NBUF, CHUNK, D), jnp.float32),
        pltpu.SemaphoreType.DMA,
        pltpu.SemaphoreType.DMA,
    ],
    compiler_params=pltpu.CompilerParams(use_tc_tiling_on_sc=False),
)
def _emb_lookup(xt_hbm, table_hbm, out_hbm, xloc, rows_v, sem_g, sem_o):
    wid = lax.axis_index("s") * NC + lax.axis_index("c")
    b0w = wid * BPW
    pltpu.sync_copy(xt_hbm.at[:, pl.ds(b0w, BPW)], xloc)

    def round_body(g, carry):
        gathers = []
        for b in range(NBUF):
            u = g * NBUF + b
            blk = u // HIST
            h = u % HIST
            c = pltpu.make_async_copy(
                table_hbm.at[xloc.at[h, pl.ds(blk * CHUNK, CHUNK)]],
                rows_v.at[b], sem_g)
            c.start()
            gathers.append(c)
        outs = []
        for b in range(NBUF):
            u = g * NBUF + b
            blk = u // HIST
            h = u % HIST
            gathers[b].wait()
            oc = pltpu.make_async_copy(
                rows_v.at[b],
                out_hbm.at[pl.ds(b0w + blk * CHUNK, CHUNK), h], sem_o)
            oc.start()
            outs.append(oc)
        for oc in outs:
            oc.wait()
        return carry

    lax.fori_loop(0, NR, round_body, 0)


def kernel(x, lut):
    xt = jnp.transpose(jnp.squeeze(x, axis=-1), (1, 0))
    return _emb_lookup(xt, lut)


# speedup vs baseline: 2.4898x; 1.3684x over previous
import functools

import jax
import jax.numpy as jnp
from jax import lax
from jax.experimental import pallas as pl
from jax.experimental.pallas import tpu as pltpu
from jax.experimental.pallas import tpu_sc as plsc

BATCH = 16384
HIST = 50
D = 32

NC = 2
NS = 16
NW = NC * NS
BPW = BATCH // NW
CHUNK = 128
NBLK = BPW // CHUNK
NUNIT = NBLK * HIST
NBUF = 4
NR = NUNIT // NBUF
TP = 129

_mesh = plsc.VectorSubcoreMesh(core_axis_name="c", subcore_axis_name="s")


@functools.partial(
    pl.kernel,
    mesh=_mesh,
    out_type=jax.ShapeDtypeStruct((HIST, D // 8, BATCH // CHUNK, 8, CHUNK),
                                  jnp.float32),
    scratch_types=[
        pltpu.VMEM((HIST, BPW), jnp.int32),
        pltpu.VMEM((NBUF, CHUNK, D), jnp.float32),
        pltpu.VMEM((NBUF, D, TP), jnp.float32),
        pltpu.SemaphoreType.DMA,
        pltpu.SemaphoreType.DMA,
    ],
    compiler_params=pltpu.CompilerParams(use_tc_tiling_on_sc=False,
                                         needs_layout_passes=False),
)
def _emb_lookup(xt_hbm, table_hbm, out_hbm, xloc, rows_v, tp_v, sem_g, sem_o):
    wid = lax.axis_index("s") * NC + lax.axis_index("c")
    b0w = wid * BPW
    pltpu.sync_copy(xt_hbm.at[:, pl.ds(b0w, BPW)], xloc)

    iota = lax.iota(jnp.int32, 16)
    row_lo = iota
    row_hi = iota + 16

    def round_body(g, carry):
        gathers = []
        for b in range(NBUF):
            u = g * NBUF + b
            blk = u // HIST
            h = u % HIST
            c = pltpu.make_async_copy(
                table_hbm.at[xloc.at[h, pl.ds(blk * CHUNK, CHUNK)]],
                rows_v.at[b], sem_g)
            c.start()
            gathers.append(c)
        outs = []
        for b in range(NBUF):
            u = g * NBUF + b
            blk = u // HIST
            h = u % HIST
            gathers[b].wait()
            gbuf = rows_v.at[b]
            tbuf = tp_v.at[b]

            def tp_body(i, _, gbuf=gbuf, tbuf=tbuf):
                for j in range(8):
                    l = i * 8 + j
                    col = row_lo * 0 + l
                    plsc.store_scatter(tbuf, [row_lo, col],
                                       gbuf[l, pl.ds(0, 16)])
                    plsc.store_scatter(tbuf, [row_hi, col],
                                       gbuf[l, pl.ds(16, 16)])
                return _

            lax.fori_loop(0, CHUNK // 8, tp_body, 0)
            for r in range(D // 8):
                oc = pltpu.make_async_copy(
                    tp_v.at[b, pl.ds(r * 8, 8), pl.ds(0, CHUNK)],
                    out_hbm.at[h, r, wid * NBLK + blk], sem_o)
                oc.start()
                outs.append(oc)
        for oc in outs:
            oc.wait()
        return carry

    lax.fori_loop(0, NR, round_body, 0)


def kernel(x, lut):
    xt = jnp.transpose(jnp.squeeze(x, axis=-1), (1, 0))
    o5 = _emb_lookup(xt, lut)
    return jnp.transpose(o5, (2, 4, 0, 1, 3)).reshape(BATCH, HIST, D)


# speedup vs baseline: 2.9363x; 1.1793x over previous
import functools

import jax
import jax.numpy as jnp
from jax import lax
from jax.experimental import pallas as pl
from jax.experimental.pallas import tpu as pltpu
from jax.experimental.pallas import tpu_sc as plsc

BATCH = 16384
HIST = 50
D = 32

NC = 2
NS = 16
NW = NC * NS
BPW = BATCH // NW
CHUNK = 128
NBLK = BPW // CHUNK
NUNIT = NBLK * HIST
NBUF = 4
NR = NUNIT // NBUF
TP = 129

_mesh = plsc.VectorSubcoreMesh(core_axis_name="c", subcore_axis_name="s")


@functools.partial(
    pl.kernel,
    mesh=_mesh,
    out_type=jax.ShapeDtypeStruct((HIST, D // 8, BATCH // CHUNK, 8, CHUNK),
                                  jnp.float32),
    scratch_types=[
        pltpu.VMEM((HIST, BPW), jnp.int32),
        pltpu.VMEM((NBUF, CHUNK, D), jnp.float32),
        pltpu.VMEM((NBUF, D, TP), jnp.float32),
        pltpu.SemaphoreType.DMA,
        pltpu.SemaphoreType.DMA,
    ],
    compiler_params=pltpu.CompilerParams(use_tc_tiling_on_sc=False,
                                         needs_layout_passes=False),
)
def _emb_lookup(xt_hbm, table_hbm, out_hbm, xloc, rows_v, tp_v, sem_g, sem_o):
    wid = lax.axis_index("s") * NC + lax.axis_index("c")
    b0w = wid * BPW
    pltpu.sync_copy(xt_hbm.at[:, pl.ds(b0w, BPW)], xloc)

    iota = lax.iota(jnp.int32, 16)
    row_lo = iota
    row_hi = iota + 16
    zerov = iota * 0

    def round_body(g, carry):
        gathers = []
        for b in range(NBUF):
            u = g * NBUF + b
            blk = u // HIST
            h = u % HIST
            c = pltpu.make_async_copy(
                table_hbm.at[xloc.at[h, pl.ds(blk * CHUNK, CHUNK)]],
                rows_v.at[b], sem_g)
            c.start()
            gathers.append(c)
        outs = []
        for b in range(NBUF):
            u = g * NBUF + b
            blk = u // HIST
            h = u % HIST
            gathers[b].wait()
            gbuf = rows_v.at[b]
            tbuf = tp_v.at[b]

            def tp_body(i, _, gbuf=gbuf, tbuf=tbuf):
                l0 = i * 8
                colv = zerov + l0
                vals = []
                for j in range(8):
                    vals.append((gbuf[l0 + j, pl.ds(0, 16)],
                                 gbuf[l0 + j, pl.ds(16, 16)]))
                for j in range(8):
                    lo, hi = vals[j]
                    col = colv + j
                    plsc.store_scatter(tbuf, [row_lo, col], lo)
                    plsc.store_scatter(tbuf, [row_hi, col], hi)
                return _

            lax.fori_loop(0, CHUNK // 8, tp_body, 0)
            for r in range(D // 8):
                oc = pltpu.make_async_copy(
                    tp_v.at[b, pl.ds(r * 8, 8), pl.ds(0, CHUNK)],
                    out_hbm.at[h, r, wid * NBLK + blk], sem_o)
                oc.start()
                outs.append(oc)
        for oc in outs:
            oc.wait()
        return carry

    lax.fori_loop(0, NR, round_body, 0)


def kernel(x, lut):
    xt = jnp.transpose(jnp.squeeze(x, axis=-1), (1, 0))
    o5 = _emb_lookup(xt, lut)
    return jnp.transpose(o5, (2, 4, 0, 1, 3)).reshape(BATCH, HIST, D)
